# Initial kernel scaffold; baseline (speedup 1.0000x reference)
import functools

import jax
import jax.numpy as jnp
from jax import lax
from jax.experimental import pallas as pl
from jax.experimental.pallas import tpu as pltpu
from jax.experimental.pallas import tpu_sc as plsc

_VOCAB = 100000
_D = 128
_SCALE = float(_D) ** 0.5

_NC = 2
_NS = 16
_NW = _NC * _NS

_B = 4096 * 200
_BPW = _B // _NW
_C = 128
_NCHUNK = _BPW // _C

_ROW_BLK = 2500


def _scale_body(lut_ref, out_ref):
    out_ref[...] = lut_ref[...] * _SCALE


def _scale_lut(lut):
    return pl.pallas_call(
        _scale_body,
        out_shape=jax.ShapeDtypeStruct(lut.shape, lut.dtype),
        grid=(_VOCAB // _ROW_BLK,),
        in_specs=[pl.BlockSpec((_ROW_BLK, _D), lambda i: (i, 0))],
        out_specs=pl.BlockSpec((_ROW_BLK, _D), lambda i: (i, 0)),
    )(lut)


def _gather_body(idx_hbm, table_hbm, out_hbm, idx_v, rows_v, sem):
    wid = lax.axis_index("s") * _NC + lax.axis_index("c")
    base = wid * _BPW
    pltpu.sync_copy(idx_hbm.at[wid], idx_v)

    def chunk(j, carry):
        pltpu.async_copy(table_hbm.at[idx_v.at[j]], rows_v, sem).wait()
        pltpu.sync_copy(rows_v, out_hbm.at[pl.ds(base + j * _C, _C)])
        return carry

    lax.fori_loop(0, _NCHUNK, chunk, 0)


_gather = functools.partial(
    pl.kernel,
    mesh=plsc.VectorSubcoreMesh(core_axis_name="c", subcore_axis_name="s"),
    out_type=jax.ShapeDtypeStruct((_B, _D), jnp.float32),
    scratch_types=[
        pltpu.VMEM((_NCHUNK, _C), jnp.int32),
        pltpu.VMEM((_C, _D), jnp.float32),
        pltpu.SemaphoreType.DMA,
    ],
)(_gather_body)


def kernel(tokens, lut):
    scaled = _scale_lut(lut)
    idx = tokens.reshape(_NW, _NCHUNK, _C).astype(jnp.int32)
    out = _gather(idx, scaled)
    return out.reshape(tokens.shape[0], tokens.shape[1], _D)


# speedup vs baseline: 5.7103x; 5.7103x over previous
import functools

import jax
import jax.numpy as jnp
from jax import lax
from jax.experimental import pallas as pl
from jax.experimental.pallas import tpu as pltpu
from jax.experimental.pallas import tpu_sc as plsc

_VOCAB = 100000
_D = 128
_SCALE = float(_D) ** 0.5

_NC = 2
_NS = 16
_NW = _NC * _NS

_B = 4096 * 200
_BPW = _B // _NW
_C = 128
_NCHUNK = _BPW // _C

_ROW_BLK = 2000


def _scale_body(lut_ref, out_ref):
    out_ref[...] = lut_ref[...] * _SCALE


def _scale_lut(lut):
    return pl.pallas_call(
        _scale_body,
        out_shape=jax.ShapeDtypeStruct(lut.shape, lut.dtype),
        grid=(_VOCAB // _ROW_BLK,),
        in_specs=[pl.BlockSpec((_ROW_BLK, _D), lambda i: (i, 0))],
        out_specs=pl.BlockSpec((_ROW_BLK, _D), lambda i: (i, 0)),
    )(lut)


def _gather_body(idx_hbm, table_hbm, out_hbm, idx_v, rows_v, sem):
    wid = lax.axis_index("s") * _NC + lax.axis_index("c")
    base = wid * _BPW
    pltpu.sync_copy(idx_hbm.at[wid], idx_v)

    def chunk(j, carry):
        pltpu.async_copy(table_hbm.at[idx_v.at[j]], rows_v, sem).wait()
        pltpu.sync_copy(rows_v, out_hbm.at[pl.ds(base + j * _C, _C)])
        return carry

    lax.fori_loop(0, _NCHUNK, chunk, 0)


_gather = functools.partial(
    pl.kernel,
    mesh=plsc.VectorSubcoreMesh(core_axis_name="c", subcore_axis_name="s"),
    out_type=jax.ShapeDtypeStruct((_B, _D), jnp.float32),
    scratch_types=[
        pltpu.VMEM((_NCHUNK, _C), jnp.int32),
        pltpu.VMEM((_C, _D), jnp.float32),
        pltpu.SemaphoreType.DMA,
    ],
)(_gather_body)


def kernel(tokens, lut):
    scaled = _scale_lut(lut)
    idx = tokens.reshape(_NW, _NCHUNK, _C).astype(jnp.int32)
    out = _gather(idx, scaled)
    return out.reshape(tokens.shape[0], tokens.shape[1], _D)


# speedup vs baseline: 7.9847x; 1.3983x over previous
import functools

import jax
import jax.numpy as jnp
from jax import lax
from jax.experimental import pallas as pl
from jax.experimental.pallas import tpu as pltpu
from jax.experimental.pallas import tpu_sc as plsc

_VOCAB = 100000
_D = 128
_SCALE = float(_D) ** 0.5

_NC = 2
_NS = 16
_NW = _NC * _NS

_B = 4096 * 200
_BPW = _B // _NW
_C = 64
_NCHUNK = _BPW // _C
_NBUF = 8
_LOOK = 4
_NR = _NCHUNK // _NBUF

_ROW_BLK = 2000


def _scale_body(lut_ref, out_ref):
    out_ref[...] = lut_ref[...] * _SCALE


def _scale_lut(lut):
    return pl.pallas_call(
        _scale_body,
        out_shape=jax.ShapeDtypeStruct(lut.shape, lut.dtype),
        grid=(_VOCAB // _ROW_BLK,),
        in_specs=[pl.BlockSpec((_ROW_BLK, _D), lambda i: (i, 0))],
        out_specs=pl.BlockSpec((_ROW_BLK, _D), lambda i: (i, 0)),
    )(lut)


def _gather_body(idx_hbm, table_hbm, out_hbm, idx_v, bufs, gsem, ssem):
    wid = lax.axis_index("s") * _NC + lax.axis_index("c")
    base = wid * _BPW
    pltpu.sync_copy(idx_hbm.at[wid], idx_v)

    def g_start(j, b):
        pltpu.async_copy(table_hbm.at[idx_v.at[j]], bufs.at[b], gsem.at[b])

    def g_wait(b):
        pltpu.make_async_copy(
            table_hbm.at[idx_v.at[0]], bufs.at[b], gsem.at[b]
        ).wait()

    def s_start(j, b):
        pltpu.async_copy(
            bufs.at[b], out_hbm.at[pl.ds(base + j * _C, _C)], ssem.at[b]
        )

    def s_wait(b):
        pltpu.make_async_copy(
            bufs.at[b], out_hbm.at[pl.ds(base, _C)], ssem.at[b]
        ).wait()

    for b in range(_LOOK):
        g_start(b, b)

    for b in range(_NBUF):
        nb = (b + _LOOK) % _NBUF
        if b >= _LOOK:
            s_wait(nb)
        g_start(b + _LOOK, nb)
        g_wait(b)
        s_start(b, b)

    def round_body(r, carry):
        j0 = r * _NBUF
        for b in range(_NBUF):
            nb = (b + _LOOK) % _NBUF
            s_wait(nb)
            g_start(j0 + b + _LOOK, nb)
            g_wait(b)
            s_start(j0 + b, b)
        return carry

    lax.fori_loop(1, _NR - 1, round_body, 0)

    j0 = (_NR - 1) * _NBUF
    for b in range(_NBUF):
        nb = (b + _LOOK) % _NBUF
        if b < _LOOK:
            s_wait(nb)
            g_start(j0 + b + _LOOK, nb)
        g_wait(b)
        s_start(j0 + b, b)

    for b in range(_NBUF):
        s_wait(b)


_gather = functools.partial(
    pl.kernel,
    mesh=plsc.VectorSubcoreMesh(core_axis_name="c", subcore_axis_name="s"),
    out_type=jax.ShapeDtypeStruct((_B, _D), jnp.float32),
    scratch_types=[
        pltpu.VMEM((_NCHUNK, _C), jnp.int32),
        pltpu.VMEM((_NBUF, _C, _D), jnp.float32),
        pltpu.SemaphoreType.DMA((_NBUF,)),
        pltpu.SemaphoreType.DMA((_NBUF,)),
    ],
)(_gather_body)


def kernel(tokens, lut):
    scaled = _scale_lut(lut)
    idx = tokens.reshape(_NW, _NCHUNK, _C).astype(jnp.int32)
    out = _gather(idx, scaled)
    return out.reshape(tokens.shape[0], tokens.shape[1], _D)


# speedup vs baseline: 8.3766x; 1.0491x over previous
import functools

import jax
import jax.numpy as jnp
from jax import lax
from jax.experimental import pallas as pl
from jax.experimental.pallas import tpu as pltpu
from jax.experimental.pallas import tpu_sc as plsc

_VOCAB = 100000
_D = 128
_SCALE = float(_D) ** 0.5

_NC = 2
_NS = 16
_NW = _NC * _NS

_B = 4096 * 200
_BPW = _B // _NW
_C = 128
_NCHUNK = _BPW // _C
_NBUF = 4
_LOOK = 2
_NR = _NCHUNK // _NBUF

_ROW_BLK = 10000


def _scale_body(lut_ref, out_ref):
    out_ref[...] = lut_ref[...] * _SCALE


def _scale_lut(lut):
    return pl.pallas_call(
        _scale_body,
        out_shape=jax.ShapeDtypeStruct(lut.shape, lut.dtype),
        grid=(_VOCAB // _ROW_BLK,),
        in_specs=[pl.BlockSpec((_ROW_BLK, _D), lambda i: (i, 0))],
        out_specs=pl.BlockSpec((_ROW_BLK, _D), lambda i: (i, 0)),
    )(lut)


def _gather_body(idx_hbm, table_hbm, out_hbm, idx_v, bufs, gsem, ssem):
    wid = lax.axis_index("s") * _NC + lax.axis_index("c")
    base = wid * _BPW
    pltpu.sync_copy(idx_hbm.at[wid], idx_v)

    def g_start(j, b):
        pltpu.async_copy(table_hbm.at[idx_v.at[j]], bufs.at[b], gsem.at[b])

    def g_wait(b):
        pltpu.make_async_copy(
            table_hbm.at[idx_v.at[0]], bufs.at[b], gsem.at[b]
        ).wait()

    def s_start(j, b):
        pltpu.async_copy(
            bufs.at[b], out_hbm.at[pl.ds(base + j * _C, _C)], ssem.at[b]
        )

    def s_wait(b):
        pltpu.make_async_copy(
            bufs.at[b], out_hbm.at[pl.ds(base, _C)], ssem.at[b]
        ).wait()

    for b in range(_LOOK):
        g_start(b, b)

    for b in range(_NBUF):
        nb = (b + _LOOK) % _NBUF
        if b >= _LOOK:
            s_wait(nb)
        g_start(b + _LOOK, nb)
        g_wait(b)
        s_start(b, b)

    def round_body(r, carry):
        j0 = r * _NBUF
        for b in range(_NBUF):
            nb = (b + _LOOK) % _NBUF
            s_wait(nb)
            g_start(j0 + b + _LOOK, nb)
            g_wait(b)
            s_start(j0 + b, b)
        return carry

    lax.fori_loop(1, _NR - 1, round_body, 0)

    j0 = (_NR - 1) * _NBUF
    for b in range(_NBUF):
        nb = (b + _LOOK) % _NBUF
        if b < _LOOK:
            s_wait(nb)
            g_start(j0 + b + _LOOK, nb)
        g_wait(b)
        s_start(j0 + b, b)

    for b in range(_NBUF):
        s_wait(b)


_gather = functools.partial(
    pl.kernel,
    mesh=plsc.VectorSubcoreMesh(core_axis_name="c", subcore_axis_name="s"),
    out_type=jax.ShapeDtypeStruct((_B, _D), jnp.float32),
    scratch_types=[
        pltpu.VMEM((_NCHUNK, _C), jnp.int32),
        pltpu.VMEM((_NBUF, _C, _D), jnp.float32),
        pltpu.SemaphoreType.DMA((_NBUF,)),
        pltpu.SemaphoreType.DMA((_NBUF,)),
    ],
)(_gather_body)


def kernel(tokens, lut):
    scaled = _scale_lut(lut)
    idx = tokens.reshape(_NW, _NCHUNK, _C).astype(jnp.int32)
    out = _gather(idx, scaled)
    return out.reshape(tokens.shape[0], tokens.shape[1], _D)


# speedup vs baseline: 8.4003x; 1.0028x over previous
import functools

import jax
import jax.numpy as jnp
from jax import lax
from jax.experimental import pallas as pl
from jax.experimental.pallas import tpu as pltpu
from jax.experimental.pallas import tpu_sc as plsc

_VOCAB = 100000
_D = 128
_SCALE = float(_D) ** 0.5

_NC = 2
_NS = 16
_NW = _NC * _NS

_B = 4096 * 200
_BPW = _B // _NW
_C = 128
_NCHUNK = _BPW // _C
_NBUF = 4
_LOOK = 2
_NR = _NCHUNK // _NBUF

_ROW_BLK = 20000


def _scale_body(lut_ref, out_ref):
    out_ref[...] = lut_ref[...] * _SCALE


def _scale_lut(lut):
    return pl.pallas_call(
        _scale_body,
        out_shape=jax.ShapeDtypeStruct(lut.shape, lut.dtype),
        grid=(_VOCAB // _ROW_BLK,),
        in_specs=[pl.BlockSpec((_ROW_BLK, _D), lambda i: (i, 0))],
        out_specs=pl.BlockSpec((_ROW_BLK, _D), lambda i: (i, 0)),
    )(lut)


def _gather_body(idx_hbm, table_hbm, out_hbm, idx_v, bufs, gsem, ssem):
    wid = lax.axis_index("s") * _NC + lax.axis_index("c")
    base = wid * _BPW
    pltpu.sync_copy(idx_hbm.at[wid], idx_v)

    def g_start(j, b):
        pltpu.async_copy(table_hbm.at[idx_v.at[j]], bufs.at[b], gsem.at[b])

    def g_wait(b):
        pltpu.make_async_copy(
            table_hbm.at[idx_v.at[0]], bufs.at[b], gsem.at[b]
        ).wait()

    def s_start(j, b):
        pltpu.async_copy(
            bufs.at[b], out_hbm.at[pl.ds(base + j * _C, _C)], ssem.at[b]
        )

    def s_wait(b):
        pltpu.make_async_copy(
            bufs.at[b], out_hbm.at[pl.ds(base, _C)], ssem.at[b]
        ).wait()

    for b in range(_LOOK):
        g_start(b, b)

    for b in range(_NBUF):
        nb = (b + _LOOK) % _NBUF
        if b >= _LOOK:
            s_wait(nb)
        g_start(b + _LOOK, nb)
        g_wait(b)
        s_start(b, b)

    def round_body(r, carry):
        j0 = r * _NBUF
        for b in range(_NBUF):
            nb = (b + _LOOK) % _NBUF
            s_wait(nb)
            g_start(j0 + b + _LOOK, nb)
            g_wait(b)
            s_start(j0 + b, b)
        return carry

    lax.fori_loop(1, _NR - 1, round_body, 0)

    j0 = (_NR - 1) * _NBUF
    for b in range(_NBUF):
        nb = (b + _LOOK) % _NBUF
        if b < _LOOK:
            s_wait(nb)
            g_start(j0 + b + _LOOK, nb)
        g_wait(b)
        s_start(j0 + b, b)

    for b in range(_NBUF):
        s_wait(b)


_gather = functools.partial(
    pl.kernel,
    mesh=plsc.VectorSubcoreMesh(core_axis_name="c", subcore_axis_name="s"),
    out_type=jax.ShapeDtypeStruct((_B, _D), jnp.float32),
    scratch_types=[
        pltpu.VMEM((_NCHUNK, _C), jnp.int32),
        pltpu.VMEM((_NBUF, _C, _D), jnp.float32),
        pltpu.SemaphoreType.DMA((_NBUF,)),
        pltpu.SemaphoreType.DMA((_NBUF,)),
    ],
)(_gather_body)


def kernel(tokens, lut):
    scaled = _scale_lut(lut)
    idx = tokens.reshape(_NW, _NCHUNK, _C).astype(jnp.int32)
    out = _gather(idx, scaled)
    return out.reshape(tokens.shape[0], tokens.shape[1], _D)
